# baseline (speedup 1.0000x reference)
import functools

import jax
import jax.numpy as jnp
from jax import lax
from jax.experimental import pallas as pl
from jax.experimental.pallas import tpu as pltpu
from jax.experimental.pallas import tpu_sc as plsc

LANES = 16
CHUNK = 16
NBUF = 4
EPS = 1e-12


_DNUMS = lax.GatherDimensionNumbers(
    offset_dims=(), collapsed_slice_dims=(0,), start_index_map=(0,))


def _shuffle(x, perm):
    return lax.gather(x, perm.reshape(LANES, 1), _DNUMS, slice_sizes=(1,),
                      mode=lax.GatherScatterMode.PROMISE_IN_BOUNDS)


def _lane_sum(x):
    iota = lax.iota(jnp.int32, LANES)
    for k in (8, 4, 2, 1):
        x = x + _shuffle(x, lax.bitwise_xor(iota, k))
    return x


def _lane_bcast(x, r):
    return _shuffle(x, lax.broadcast_in_dim(r, (LANES,), ()))


@functools.lru_cache(maxsize=None)
def _make_sc_kernel(n_tokens, seq, hidden):
    info = plsc.get_sparse_core_info()
    n_workers = info.num_cores * info.num_subcores
    assert n_tokens % (n_workers * seq) == 0, "each worker owns whole sequences"
    tok_per_w = n_tokens // n_workers
    seq_per_w = tok_per_w // seq
    n_chunks = tok_per_w // CHUNK
    assert n_chunks % NBUF == 0
    assert seq % CHUNK == 0 and hidden % LANES == 0
    hchunks = hidden // LANES
    words_per_w = n_chunks
    inv_h = 1.0 / hidden

    @functools.partial(
        pl.kernel,
        out_type=jax.ShapeDtypeStruct((n_tokens, hidden), jnp.float32),
        mesh=plsc.VectorSubcoreMesh(core_axis_name="c", subcore_axis_name="s"),
        scratch_types=(
            [pltpu.VMEM((tok_per_w,), jnp.int32),
             pltpu.VMEM((words_per_w,), jnp.int32),
             pltpu.VMEM((CHUNK, hidden), jnp.float32)]
            + [pltpu.VMEM((CHUNK, hidden), jnp.float32)] * NBUF
            + [pltpu.VMEM((hidden,), jnp.float32)] * 3
            + [pltpu.SemaphoreType.DMA] * (2 * NBUF)
        ),
    )
    def sc_kernel(ids_hbm, tpk_hbm, word_hbm, pose0_hbm, dt_hbm, gamma_hbm,
                  beta_hbm, out_hbm, ids_v, tpk_v, pos_v, rb0, rb1, rb2, rb3,
                  gamma_v, beta_v, dt_v, g0, g1, g2, g3, w0, w1, w2, w3):
        bufs = (rb0, rb1, rb2, rb3)
        gsems = (g0, g1, g2, g3)
        wsems = (w0, w1, w2, w3)
        wid = lax.axis_index("s") * info.num_cores + lax.axis_index("c")
        tok0 = wid * tok_per_w
        pltpu.sync_copy(ids_hbm.at[pl.ds(tok0, tok_per_w)], ids_v)
        pltpu.sync_copy(tpk_hbm.at[pl.ds(wid * words_per_w, words_per_w)], tpk_v)
        pltpu.sync_copy(gamma_hbm, gamma_v)
        pltpu.sync_copy(beta_hbm, beta_v)
        pltpu.sync_copy(dt_hbm, dt_v)

        def chunk_off(t):
            b = lax.rem(t, seq_per_w)
            j = t // seq_per_w
            return b, j, b * seq + j * CHUNK

        def issue_gather(t, ph):
            _, _, off = chunk_off(t)
            pltpu.async_copy(word_hbm.at[ids_v.at[pl.ds(off, CHUNK)]],
                             bufs[ph], gsems[ph])

        def wait_gather(ph):
            pltpu.make_async_copy(word_hbm.at[pl.ds(0, CHUNK)], bufs[ph],
                                  gsems[ph]).wait()

        def wait_write(ph):
            pltpu.make_async_copy(bufs[ph], out_hbm.at[pl.ds(0, CHUNK)],
                                  wsems[ph]).wait()

        issue_gather(jnp.int32(0), 0)
        issue_gather(jnp.int32(1), 1)

        def phase_body(t, ph):
            b, j, off = chunk_off(t)
            buf = bufs[ph]
            pl.when(lax.rem(t, seq_per_w) == 0)(
                lambda: pltpu.sync_copy(pose0_hbm.at[pl.ds(j * CHUNK, CHUNK)],
                                        pos_v))
            wait_gather(ph)
            pl.when(t >= 2)(lambda: wait_write((ph + 2) % NBUF))
            pl.when(t + 2 < n_chunks)(
                lambda: issue_gather(t + 2, (ph + 2) % NBUF))

            iota = lax.iota(jnp.int32, LANES)
            twords = tpk_v[pl.ds((t // LANES) * LANES, LANES)]
            tword = _lane_bcast(twords, lax.rem(t, LANES))
            tbits = lax.shift_right_logical(tword, iota) & 1
            tf = tbits.astype(jnp.float32)

            def row_body(r, rcarry):
                t_r = _lane_bcast(tf, r)
                acc = jnp.zeros((LANES,), jnp.float32)
                accsq = jnp.zeros((LANES,), jnp.float32)
                for c in range(hchunks):
                    sl = pl.ds(c * LANES, LANES)
                    x = buf[r, sl] + pos_v[r, sl] + t_r * dt_v[sl]
                    buf[r, sl] = x
                    acc = acc + x
                    accsq = accsq + x * x
                mean_v = _lane_sum(acc) * inv_h
                var_v = _lane_sum(accsq) * inv_h - mean_v * mean_v
                v = var_v + EPS
                bits = lax.bitcast_convert_type(v, jnp.int32)
                ones = jnp.full((LANES,), 1, jnp.int32)
                bits = 0x5F3759DF - lax.shift_right_logical(bits, ones)
                y = lax.bitcast_convert_type(bits, jnp.float32)
                half = v * 0.5
                for _ in range(3):
                    y = y * (1.5 - half * y * y)
                for c in range(hchunks):
                    sl = pl.ds(c * LANES, LANES)
                    xhat = (buf[r, sl] - mean_v) * y
                    buf[r, sl] = xhat * gamma_v[sl] + beta_v[sl]
                return rcarry

            lax.fori_loop(0, CHUNK, row_body, 0)
            pltpu.async_copy(buf, out_hbm.at[pl.ds(tok0 + off, CHUNK)],
                             wsems[ph])

        def ring_body(i, carry):
            for ph in range(NBUF):
                phase_body(i * NBUF + ph, ph)
            return carry

        lax.fori_loop(0, n_chunks // NBUF, ring_body, 0)
        wait_write(NBUF - 2)
        wait_write(NBUF - 1)

    return sc_kernel


def kernel(input_ids, token_type_ids, word_embeddings, position_embeddings,
           token_type_embeddings, gamma, beta):
    batch, seq = input_ids.shape
    hidden = word_embeddings.shape[1]
    n_tokens = batch * seq
    ids = input_ids.reshape(-1).astype(jnp.int32)
    tids = token_type_ids.reshape(-1).astype(jnp.int32)
    info = plsc.get_sparse_core_info()
    n_workers = info.num_cores * info.num_subcores
    seq_per_w = n_tokens // seq // n_workers
    slots = seq // CHUNK
    tpk = (tids.reshape(n_workers, seq_per_w, slots, LANES)
           * (1 << jnp.arange(LANES, dtype=jnp.int32))).sum(
               axis=-1, dtype=jnp.int32).transpose(0, 2, 1).reshape(-1)
    pose0 = position_embeddings[:seq] + token_type_embeddings[0][None, :]
    dt = token_type_embeddings[1] - token_type_embeddings[0]
    sc = _make_sc_kernel(n_tokens, seq, hidden)
    out = sc(ids, tpk, word_embeddings, pose0, dt, gamma, beta)
    return out.reshape(batch, seq, hidden)
